# baseline (speedup 1.0000x reference)
import functools

import jax
import jax.numpy as jnp
from jax import lax
from jax.experimental import pallas as pl
from jax.experimental.pallas import tpu as pltpu
from jax.experimental.pallas import tpu_sc as plsc

B = 128
L = 375
LPAD = 384
V = 16384
D = 1280

NC = 2
NS = 16
NW = NC * NS
BPW = B // NW

LEN_COL = 376
GCHUNK = 64
NCHUNK = LPAD // GCHUNK
NLB = GCHUNK // 16
NBUF = 3

DBLK = 128
NDBLK = D // DBLK
NDV = DBLK // 16
ITEMS = BPW * NDBLK


def _body(cb_hbm, tok_hbm, out_hbm, tok_v, gbig, sbig,
          sem_g0, sem_g1, sem_g2, sem_w0, sem_w1):
    wid = lax.axis_index("s") * NC + lax.axis_index("c")
    iota = lax.iota(jnp.int32, 16)
    zeros16 = jnp.zeros((16,), jnp.int32)
    zf16 = jnp.zeros((16,), jnp.float32)

    gsems = ((0, sem_g0), (1, sem_g1), (2, sem_g2))

    def gslice(par):
        return gbig.at[pl.ds(pl.multiple_of(par * GCHUNK, GCHUNK), GCHUNK)]

    def issue_gather(c, d0, par, sem):
        pltpu.async_copy(
            cb_hbm.at[tok_v.at[pl.ds(pl.multiple_of(c * GCHUNK, GCHUNK),
                                     GCHUNK)],
                      pl.ds(d0, DBLK)],
            gslice(par), sem)

    def wait_gather(d0, par, sem):
        pltpu.make_async_copy(
            cb_hbm.at[tok_v.at[pl.ds(0, GCHUNK)], pl.ds(d0, DBLK)],
            gslice(par), sem).wait()

    def transpose_chunk(c, par, spar, len16):
        gbase = par * GCHUNK
        spar16 = zeros16 + spar

        def lb_loop(lb, _):
            row16 = (gbase + lb * 16) + iota
            col16 = (c * GCHUNK + lb * 16) + iota
            lmask = col16 < len16

            @plsc.parallel_loop(0, 16, 1, unroll=4)
            def k_loop(k):
                rot = (iota + k) & 15
                for dv in range(NDV):
                    dloc = dv * 16 + rot
                    vec = plsc.load_gather(gbig, [row16, dloc])
                    val = jnp.where(lmask, vec, zf16)
                    plsc.store_scatter(sbig, [spar16, dloc, col16], val)

            return 0

        lax.fori_loop(0, NLB, lb_loop, 0)

    def do_item(t, _):
        bi = t // NDBLK
        b = wid * BPW + bi
        d0 = pl.multiple_of((t % NDBLK) * DBLK, DBLK)
        d0_next = pl.multiple_of(((t + 1) % NDBLK) * DBLK, DBLK)

        lvec = tok_v[pl.ds(LEN_COL - LEN_COL % 16, 16)]
        len16 = zeros16 + lvec[LEN_COL % 16]

        spar = t % 2
        for wp, sem_w in ((0, sem_w0), (1, sem_w1)):
            @pl.when(jnp.logical_and(spar == wp, t >= 2))
            def _():
                pltpu.make_async_copy(
                    sbig.at[wp], out_hbm.at[b, pl.ds(d0, DBLK), :],
                    sem_w).wait()

        def chunk_body(c, _):
            par = c % NBUF
            nxt2 = (c + 2) % NBUF
            @pl.when(c + 2 < NCHUNK)
            def _():
                for gp, sem in gsems:
                    @pl.when(nxt2 == gp)
                    def _():
                        issue_gather(c + 2, d0, gp, sem)

            @pl.when(jnp.logical_and(c + 2 >= NCHUNK,
                                     (t + 1) % NDBLK != 0))
            def _():
                for gp, sem in gsems:
                    @pl.when(nxt2 == gp)
                    def _():
                        issue_gather(c + 2 - NCHUNK, d0_next, gp, sem)

            for gp, sem in gsems:
                @pl.when(par == gp)
                def _():
                    wait_gather(d0, gp, sem)

            transpose_chunk(c, par, spar, len16)
            return 0

        lax.fori_loop(0, NCHUNK, chunk_body, 0)

        for wp, sem_w in ((0, sem_w0), (1, sem_w1)):
            @pl.when(spar == wp)
            def _():
                pltpu.async_copy(
                    sbig.at[wp], out_hbm.at[b, pl.ds(d0, DBLK), :], sem_w)

        @pl.when(jnp.logical_and((t + 1) % NDBLK == 0, t + 1 < ITEMS))
        def _():
            pltpu.sync_copy(tok_hbm.at[b + 1], tok_v)
            issue_gather(0, 0, 0, sem_g0)
            issue_gather(1, 0, 1, sem_g1)
        return 0

    pltpu.sync_copy(tok_hbm.at[wid * BPW], tok_v)
    issue_gather(0, 0, 0, sem_g0)
    issue_gather(1, 0, 1, sem_g1)
    lax.fori_loop(0, ITEMS, do_item, 0)

    b_last = wid * BPW + BPW - 1
    pltpu.make_async_copy(
        sbig.at[0], out_hbm.at[b_last, pl.ds(0, DBLK), :], sem_w0).wait()
    pltpu.make_async_copy(
        sbig.at[1], out_hbm.at[b_last, pl.ds(0, DBLK), :], sem_w1).wait()


@functools.partial(jax.jit, donate_argnums=())
def _run(codebook, tokens_pad):
    mesh = plsc.VectorSubcoreMesh(core_axis_name="c", subcore_axis_name="s")
    k = pl.kernel(
        _body,
        out_type=jax.ShapeDtypeStruct((B, D, L), jnp.float32),
        mesh=mesh,
        compiler_params=pltpu.CompilerParams(
            use_tc_tiling_on_sc=True, needs_layout_passes=False),
        scratch_types=[
            pltpu.VMEM((LPAD,), jnp.int32),
            pltpu.VMEM((NBUF * GCHUNK, DBLK), jnp.float32),
            pltpu.VMEM((2, DBLK, L), jnp.float32),
            pltpu.SemaphoreType.DMA,
            pltpu.SemaphoreType.DMA,
            pltpu.SemaphoreType.DMA,
            pltpu.SemaphoreType.DMA,
            pltpu.SemaphoreType.DMA,
        ],
    )
    return k(codebook, tokens_pad)


def kernel(audio_tokens, output_lengths, codebook):
    tokens_pad = jnp.pad(audio_tokens, ((0, 0), (0, LPAD - L)))
    tokens_pad = tokens_pad.at[:, LEN_COL].set(output_lengths)
    out = _run(codebook, tokens_pad)
    return (out, output_lengths)


# speedup vs baseline: 1.0481x; 1.0481x over previous
import functools

import jax
import jax.numpy as jnp
from jax import lax
from jax.experimental import pallas as pl
from jax.experimental.pallas import tpu as pltpu
from jax.experimental.pallas import tpu_sc as plsc

B = 128
L = 375
LPAD = 384
V = 16384
D = 1280

NC = 2
NS = 16
NW = NC * NS
BPW = B // NW

LEN_COL = 376
GCHUNK = 48
NCHUNK = LPAD // GCHUNK
NLB = GCHUNK // 16
NBUF = 4

DBLK = 128
NDBLK = D // DBLK
NDV = DBLK // 16
ITEMS = BPW * NDBLK


def _body(cb_hbm, tok_hbm, out_hbm, tok_v, gbig, sbig,
          sem_g0, sem_g1, sem_g2, sem_g3, sem_w0, sem_w1):
    wid = lax.axis_index("s") * NC + lax.axis_index("c")
    iota = lax.iota(jnp.int32, 16)
    zeros16 = jnp.zeros((16,), jnp.int32)
    zf16 = jnp.zeros((16,), jnp.float32)

    gsems = ((0, sem_g0), (1, sem_g1), (2, sem_g2), (3, sem_g3))

    def gslice(par):
        return gbig.at[pl.ds(pl.multiple_of(par * GCHUNK, GCHUNK), GCHUNK)]

    def issue_gather(c, d0, par, sem):
        pltpu.async_copy(
            cb_hbm.at[tok_v.at[pl.ds(pl.multiple_of(c * GCHUNK, GCHUNK),
                                     GCHUNK)],
                      pl.ds(d0, DBLK)],
            gslice(par), sem)

    def wait_gather(d0, par, sem):
        pltpu.make_async_copy(
            cb_hbm.at[tok_v.at[pl.ds(0, GCHUNK)], pl.ds(d0, DBLK)],
            gslice(par), sem).wait()

    def transpose_chunk(c, par, spar, len16):
        gbase = par * GCHUNK
        spar16 = zeros16 + spar

        def lb_loop(lb, _):
            row16 = (gbase + lb * 16) + iota
            col16 = (c * GCHUNK + lb * 16) + iota
            lmask = col16 < len16

            @plsc.parallel_loop(0, 16, 1, unroll=2)
            def k_loop(k):
                rot = (iota + k) & 15
                for dv in range(NDV):
                    dloc = dv * 16 + rot
                    vec = plsc.load_gather(gbig, [row16, dloc])
                    val = jnp.where(lmask, vec, zf16)
                    plsc.store_scatter(sbig, [spar16, dloc, col16], val)

            return 0

        lax.fori_loop(0, NLB, lb_loop, 0)

    def do_item(t, _):
        bi = t // NDBLK
        b = wid * BPW + bi
        d0 = pl.multiple_of((t % NDBLK) * DBLK, DBLK)
        d0_next = pl.multiple_of(((t + 1) % NDBLK) * DBLK, DBLK)

        lvec = tok_v[pl.ds(LEN_COL - LEN_COL % 16, 16)]
        len16 = zeros16 + lvec[LEN_COL % 16]

        spar = t % 2
        for wp, sem_w in ((0, sem_w0), (1, sem_w1)):
            @pl.when(jnp.logical_and(spar == wp, t >= 2))
            def _():
                pltpu.make_async_copy(
                    sbig.at[wp], out_hbm.at[b, pl.ds(d0, DBLK), :],
                    sem_w).wait()

        def chunk_body(c, _):
            par = c % NBUF
            nxt3 = (c + 3) % NBUF
            @pl.when(c + 3 < NCHUNK)
            def _():
                for gp, sem in gsems:
                    @pl.when(nxt3 == gp)
                    def _():
                        issue_gather(c + 3, d0, gp, sem)

            @pl.when(jnp.logical_and(c + 3 >= NCHUNK,
                                     (t + 1) % NDBLK != 0))
            def _():
                for gp, sem in gsems:
                    @pl.when(nxt3 == gp)
                    def _():
                        issue_gather(c + 3 - NCHUNK, d0_next, gp, sem)

            for gp, sem in gsems:
                @pl.when(par == gp)
                def _():
                    wait_gather(d0, gp, sem)

            transpose_chunk(c, par, spar, len16)
            return 0

        lax.fori_loop(0, NCHUNK, chunk_body, 0)

        for wp, sem_w in ((0, sem_w0), (1, sem_w1)):
            @pl.when(spar == wp)
            def _():
                pltpu.async_copy(
                    sbig.at[wp], out_hbm.at[b, pl.ds(d0, DBLK), :], sem_w)

        @pl.when(jnp.logical_and((t + 1) % NDBLK == 0, t + 1 < ITEMS))
        def _():
            pltpu.sync_copy(tok_hbm.at[b + 1], tok_v)
            issue_gather(0, 0, 0, sem_g0)
            issue_gather(1, 0, 1, sem_g1)
            issue_gather(2, 0, 2, sem_g2)
        return 0

    pltpu.sync_copy(tok_hbm.at[wid * BPW], tok_v)
    issue_gather(0, 0, 0, sem_g0)
    issue_gather(1, 0, 1, sem_g1)
    issue_gather(2, 0, 2, sem_g2)
    lax.fori_loop(0, ITEMS, do_item, 0)

    b_last = wid * BPW + BPW - 1
    pltpu.make_async_copy(
        sbig.at[0], out_hbm.at[b_last, pl.ds(0, DBLK), :], sem_w0).wait()
    pltpu.make_async_copy(
        sbig.at[1], out_hbm.at[b_last, pl.ds(0, DBLK), :], sem_w1).wait()


@functools.partial(jax.jit, donate_argnums=())
def _run(codebook, tokens_pad):
    mesh = plsc.VectorSubcoreMesh(core_axis_name="c", subcore_axis_name="s")
    k = pl.kernel(
        _body,
        out_type=jax.ShapeDtypeStruct((B, D, L), jnp.float32),
        mesh=mesh,
        compiler_params=pltpu.CompilerParams(
            use_tc_tiling_on_sc=True, needs_layout_passes=False),
        scratch_types=[
            pltpu.VMEM((LPAD,), jnp.int32),
            pltpu.VMEM((NBUF * GCHUNK, DBLK), jnp.float32),
            pltpu.VMEM((2, DBLK, L), jnp.float32),
            pltpu.SemaphoreType.DMA,
            pltpu.SemaphoreType.DMA,
            pltpu.SemaphoreType.DMA,
            pltpu.SemaphoreType.DMA,
            pltpu.SemaphoreType.DMA,
            pltpu.SemaphoreType.DMA,
        ],
    )
    return k(codebook, tokens_pad)


def kernel(audio_tokens, output_lengths, codebook):
    tokens_pad = jnp.pad(audio_tokens, ((0, 0), (0, LPAD - L)))
    tokens_pad = tokens_pad.at[:, LEN_COL].set(output_lengths)
    out = _run(codebook, tokens_pad)
    return (out, output_lengths)


# speedup vs baseline: 1.0578x; 1.0093x over previous
import functools

import jax
import jax.numpy as jnp
from jax import lax
from jax.experimental import pallas as pl
from jax.experimental.pallas import tpu as pltpu
from jax.experimental.pallas import tpu_sc as plsc

B = 128
L = 375
LPAD = 384
V = 16384
D = 1280

NC = 2
NS = 16
NW = NC * NS
BPW = B // NW

LEN_COL = 376
GCHUNK = 32
NCHUNK = LPAD // GCHUNK
NLB = GCHUNK // 16
NBUF = 6

DBLK = 128
NDBLK = D // DBLK
NDV = DBLK // 16
ITEMS = BPW * NDBLK


def _body(cb_hbm, tok_hbm, out_hbm, tok_v, gbig, sbig,
          sem_g0, sem_g1, sem_g2, sem_g3, sem_g4, sem_g5, sem_w0, sem_w1):
    wid = lax.axis_index("s") * NC + lax.axis_index("c")
    iota = lax.iota(jnp.int32, 16)
    zeros16 = jnp.zeros((16,), jnp.int32)
    zf16 = jnp.zeros((16,), jnp.float32)

    gsems = ((0, sem_g0), (1, sem_g1), (2, sem_g2), (3, sem_g3), (4, sem_g4), (5, sem_g5))

    def gslice(par):
        return gbig.at[pl.ds(pl.multiple_of(par * GCHUNK, GCHUNK), GCHUNK)]

    def issue_gather(c, d0, par, sem):
        pltpu.async_copy(
            cb_hbm.at[tok_v.at[pl.ds(pl.multiple_of(c * GCHUNK, GCHUNK),
                                     GCHUNK)],
                      pl.ds(d0, DBLK)],
            gslice(par), sem)

    def wait_gather(d0, par, sem):
        pltpu.make_async_copy(
            cb_hbm.at[tok_v.at[pl.ds(0, GCHUNK)], pl.ds(d0, DBLK)],
            gslice(par), sem).wait()

    def transpose_chunk(c, par, spar, len16):
        gbase = par * GCHUNK
        spar16 = zeros16 + spar

        def lb_loop(lb, _):
            row16 = (gbase + lb * 16) + iota
            col16 = (c * GCHUNK + lb * 16) + iota
            lmask = col16 < len16

            @plsc.parallel_loop(0, 16, 1, unroll=2)
            def k_loop(k):
                rot = (iota + k) & 15
                for dv in range(NDV):
                    dloc = dv * 16 + rot
                    vec = plsc.load_gather(gbig, [row16, dloc])
                    val = jnp.where(lmask, vec, zf16)
                    plsc.store_scatter(sbig, [spar16, dloc, col16], val)

            return 0

        lax.fori_loop(0, NLB, lb_loop, 0)

    def do_item(t, _):
        bi = t // NDBLK
        b = wid * BPW + bi
        d0 = pl.multiple_of((t % NDBLK) * DBLK, DBLK)
        d0_next = pl.multiple_of(((t + 1) % NDBLK) * DBLK, DBLK)

        lvec = tok_v[pl.ds(LEN_COL - LEN_COL % 16, 16)]
        len16 = zeros16 + lvec[LEN_COL % 16]

        spar = t % 2
        for wp, sem_w in ((0, sem_w0), (1, sem_w1)):
            @pl.when(jnp.logical_and(spar == wp, t >= 2))
            def _():
                pltpu.make_async_copy(
                    sbig.at[wp], out_hbm.at[b, pl.ds(d0, DBLK), :],
                    sem_w).wait()

        def chunk_body(c, _):
            par = c % NBUF
            nxt3 = (c + 5) % NBUF
            @pl.when(c + 5 < NCHUNK)
            def _():
                for gp, sem in gsems:
                    @pl.when(nxt3 == gp)
                    def _():
                        issue_gather(c + 5, d0, gp, sem)

            @pl.when(jnp.logical_and(c + 5 >= NCHUNK,
                                     (t + 1) % NDBLK != 0))
            def _():
                for gp, sem in gsems:
                    @pl.when(nxt3 == gp)
                    def _():
                        issue_gather(c + 5 - NCHUNK, d0_next, gp, sem)

            for gp, sem in gsems:
                @pl.when(par == gp)
                def _():
                    wait_gather(d0, gp, sem)

            transpose_chunk(c, par, spar, len16)
            return 0

        lax.fori_loop(0, NCHUNK, chunk_body, 0)

        for wp, sem_w in ((0, sem_w0), (1, sem_w1)):
            @pl.when(spar == wp)
            def _():
                pltpu.async_copy(
                    sbig.at[wp], out_hbm.at[b, pl.ds(d0, DBLK), :], sem_w)

        @pl.when(jnp.logical_and((t + 1) % NDBLK == 0, t + 1 < ITEMS))
        def _():
            pltpu.sync_copy(tok_hbm.at[b + 1], tok_v)
            for gi, (gp, sem) in enumerate(gsems[:NBUF - 1]):
                issue_gather(gi, 0, gp, sem)
        return 0

    pltpu.sync_copy(tok_hbm.at[wid * BPW], tok_v)
    for gi, (gp, sem) in enumerate(gsems[:NBUF - 1]):
        issue_gather(gi, 0, gp, sem)
    lax.fori_loop(0, ITEMS, do_item, 0)

    b_last = wid * BPW + BPW - 1
    pltpu.make_async_copy(
        sbig.at[0], out_hbm.at[b_last, pl.ds(0, DBLK), :], sem_w0).wait()
    pltpu.make_async_copy(
        sbig.at[1], out_hbm.at[b_last, pl.ds(0, DBLK), :], sem_w1).wait()


@functools.partial(jax.jit, donate_argnums=())
def _run(codebook, tokens_pad):
    mesh = plsc.VectorSubcoreMesh(core_axis_name="c", subcore_axis_name="s")
    k = pl.kernel(
        _body,
        out_type=jax.ShapeDtypeStruct((B, D, L), jnp.float32),
        mesh=mesh,
        compiler_params=pltpu.CompilerParams(
            use_tc_tiling_on_sc=True, needs_layout_passes=False),
        scratch_types=[
            pltpu.VMEM((LPAD,), jnp.int32),
            pltpu.VMEM((NBUF * GCHUNK, DBLK), jnp.float32),
            pltpu.VMEM((2, DBLK, L), jnp.float32),
            pltpu.SemaphoreType.DMA,
            pltpu.SemaphoreType.DMA,
            pltpu.SemaphoreType.DMA,
            pltpu.SemaphoreType.DMA,
            pltpu.SemaphoreType.DMA,
            pltpu.SemaphoreType.DMA,
            pltpu.SemaphoreType.DMA,
            pltpu.SemaphoreType.DMA,
        ],
    )
    return k(codebook, tokens_pad)


def kernel(audio_tokens, output_lengths, codebook):
    tokens_pad = jnp.pad(audio_tokens, ((0, 0), (0, LPAD - L)))
    tokens_pad = tokens_pad.at[:, LEN_COL].set(output_lengths)
    out = _run(codebook, tokens_pad)
    return (out, output_lengths)


# speedup vs baseline: 1.2068x; 1.1409x over previous
import functools

import jax
import jax.numpy as jnp
from jax import lax
from jax.experimental import pallas as pl
from jax.experimental.pallas import tpu as pltpu
from jax.experimental.pallas import tpu_sc as plsc

B = 128
L = 375
LPAD = 384
V = 16384
D = 1280

NC = 2
NS = 16
NW = NC * NS
BPW = B // NW

LEN_COL = 376
GCHUNK = 32
NCHUNK = LPAD // GCHUNK
NLB = GCHUNK // 16
NBUF = 6

DBLK = 128
NDBLK = D // DBLK
NDV = DBLK // 16
ITEMS = BPW * NDBLK


def _body(cb_hbm, tok_hbm, out_hbm, tok_v, gbig, sbig,
          sem_g0, sem_g1, sem_g2, sem_g3, sem_g4, sem_g5, sem_w0, sem_w1):
    wid = lax.axis_index("s") * NC + lax.axis_index("c")
    iota = lax.iota(jnp.int32, 16)
    zeros16 = jnp.zeros((16,), jnp.int32)
    zf16 = jnp.zeros((16,), jnp.float32)

    gsems = ((0, sem_g0), (1, sem_g1), (2, sem_g2), (3, sem_g3), (4, sem_g4), (5, sem_g5))

    def gslice(par):
        return gbig.at[pl.ds(pl.multiple_of(par * GCHUNK, GCHUNK), GCHUNK)]

    def issue_gather(c, d0, par, sem):
        pltpu.async_copy(
            cb_hbm.at[tok_v.at[pl.ds(pl.multiple_of(c * GCHUNK, GCHUNK),
                                     GCHUNK)],
                      pl.ds(d0, DBLK)],
            gslice(par), sem)

    def wait_gather(d0, par, sem):
        pltpu.make_async_copy(
            cb_hbm.at[tok_v.at[pl.ds(0, GCHUNK)], pl.ds(d0, DBLK)],
            gslice(par), sem).wait()

    def transpose_chunk(c, par, spar, len16):
        gbase = par * GCHUNK
        spar16 = zeros16 + spar

        def lb_loop(lb, _):
            row16 = (gbase + lb * 16) + iota
            col16 = (c * GCHUNK + lb * 16) + iota
            lmask = col16 < len16

            @plsc.parallel_loop(0, 16, 1, unroll=2)
            def k_loop(k):
                rot = (iota + k) & 15
                for dv in range(NDV):
                    dloc = dv * 16 + rot
                    vec = plsc.load_gather(gbig, [row16, dloc])
                    val = jnp.where(lmask, vec, zf16)
                    plsc.store_scatter(sbig, [spar16, dloc, col16], val)

            return 0

        lax.fori_loop(0, NLB, lb_loop, 0)

    def do_item(t, _):
        bi = t // NDBLK
        b = wid * BPW + bi
        d0 = pl.multiple_of((t % NDBLK) * DBLK, DBLK)
        d0_next = pl.multiple_of(((t + 1) % NDBLK) * DBLK, DBLK)

        lvec = tok_v[pl.ds(LEN_COL - LEN_COL % 16, 16)]
        len_s = lvec[LEN_COL % 16]
        len16 = zeros16 + len_s

        spar = t % 2
        for wp, sem_w in ((0, sem_w0), (1, sem_w1)):
            @pl.when(jnp.logical_and(spar == wp, t >= 2))
            def _():
                pltpu.make_async_copy(
                    sbig.at[wp], out_hbm.at[b, pl.ds(d0, DBLK), :],
                    sem_w).wait()

        def chunk_body(c, _):
            par = c % NBUF
            nxt3 = (c + 5) % NBUF
            @pl.when(jnp.logical_and(c + 5 < NCHUNK,
                                     (c + 5) * GCHUNK < len_s))
            def _():
                for gp, sem in gsems:
                    @pl.when(nxt3 == gp)
                    def _():
                        issue_gather(c + 5, d0, gp, sem)

            @pl.when(jnp.logical_and(
                c + 5 >= NCHUNK,
                jnp.logical_and((t + 1) % NDBLK != 0,
                                (c + 5 - NCHUNK) * GCHUNK < len_s)))
            def _():
                for gp, sem in gsems:
                    @pl.when(nxt3 == gp)
                    def _():
                        issue_gather(c + 5 - NCHUNK, d0_next, gp, sem)

            @pl.when(c * GCHUNK < len_s)
            def _():
                for gp, sem in gsems:
                    @pl.when(par == gp)
                    def _():
                        wait_gather(d0, gp, sem)
                transpose_chunk(c, par, spar, len16)

            @pl.when(c * GCHUNK >= len_s)
            def _():
                col0 = pl.multiple_of(c * GCHUNK, GCHUNK)

                def z_loop(d, _):
                    sbig[spar, d, pl.ds(col0, 16)] = zf16
                    sbig[spar, d, pl.ds(col0 + 16, 16)] = zf16
                    return 0

                lax.fori_loop(0, DBLK, z_loop, 0)
            return 0

        lax.fori_loop(0, NCHUNK, chunk_body, 0)

        for wp, sem_w in ((0, sem_w0), (1, sem_w1)):
            @pl.when(spar == wp)
            def _():
                pltpu.async_copy(
                    sbig.at[wp], out_hbm.at[b, pl.ds(d0, DBLK), :], sem_w)

        @pl.when(jnp.logical_and((t + 1) % NDBLK == 0, t + 1 < ITEMS))
        def _():
            pltpu.sync_copy(tok_hbm.at[b + 1], tok_v)
            lv2 = tok_v[pl.ds(LEN_COL - LEN_COL % 16, 16)]
            ln2 = lv2[LEN_COL % 16]
            for gi, (gp, sem) in enumerate(gsems[:NBUF - 1]):
                @pl.when(gi * GCHUNK < ln2)
                def _():
                    issue_gather(gi, 0, gp, sem)
        return 0

    pltpu.sync_copy(tok_hbm.at[wid * BPW], tok_v)
    lv0 = tok_v[pl.ds(LEN_COL - LEN_COL % 16, 16)]
    ln0 = lv0[LEN_COL % 16]
    for gi, (gp, sem) in enumerate(gsems[:NBUF - 1]):
        @pl.when(gi * GCHUNK < ln0)
        def _():
            issue_gather(gi, 0, gp, sem)
    lax.fori_loop(0, ITEMS, do_item, 0)

    b_last = wid * BPW + BPW - 1
    pltpu.make_async_copy(
        sbig.at[0], out_hbm.at[b_last, pl.ds(0, DBLK), :], sem_w0).wait()
    pltpu.make_async_copy(
        sbig.at[1], out_hbm.at[b_last, pl.ds(0, DBLK), :], sem_w1).wait()


@functools.partial(jax.jit, donate_argnums=())
def _run(codebook, tokens_pad):
    mesh = plsc.VectorSubcoreMesh(core_axis_name="c", subcore_axis_name="s")
    k = pl.kernel(
        _body,
        out_type=jax.ShapeDtypeStruct((B, D, L), jnp.float32),
        mesh=mesh,
        compiler_params=pltpu.CompilerParams(
            use_tc_tiling_on_sc=True, needs_layout_passes=False),
        scratch_types=[
            pltpu.VMEM((LPAD,), jnp.int32),
            pltpu.VMEM((NBUF * GCHUNK, DBLK), jnp.float32),
            pltpu.VMEM((2, DBLK, L), jnp.float32),
            pltpu.SemaphoreType.DMA,
            pltpu.SemaphoreType.DMA,
            pltpu.SemaphoreType.DMA,
            pltpu.SemaphoreType.DMA,
            pltpu.SemaphoreType.DMA,
            pltpu.SemaphoreType.DMA,
            pltpu.SemaphoreType.DMA,
            pltpu.SemaphoreType.DMA,
        ],
    )
    return k(codebook, tokens_pad)


def kernel(audio_tokens, output_lengths, codebook):
    tokens_pad = jnp.pad(audio_tokens, ((0, 0), (0, LPAD - L)))
    tokens_pad = tokens_pad.at[:, LEN_COL].set(output_lengths)
    out = _run(codebook, tokens_pad)
    return (out, output_lengths)


# speedup vs baseline: 1.4329x; 1.1873x over previous
import functools

import jax
import jax.numpy as jnp
from jax import lax
from jax.experimental import pallas as pl
from jax.experimental.pallas import tpu as pltpu
from jax.experimental.pallas import tpu_sc as plsc

B = 128
L = 375
LPAD = 384
V = 16384
D = 1280

NC = 2
NS = 16
NW = NC * NS
BPW = B // NW

LEN_COL = 376
GCHUNK = 32
NCHUNK = LPAD // GCHUNK
NLB = GCHUNK // 16
NBUF = 6

DBLK = 128
NDBLK = D // DBLK
NDV = DBLK // 16
ITEMS = BPW * NDBLK


def _body(cb_hbm, tok_hbm, out_hbm, tok_v, gbig, sbig,
          sem_g0, sem_g1, sem_g2, sem_g3, sem_g4, sem_g5, sem_w0, sem_w1):
    wid = lax.axis_index("s") * NC + lax.axis_index("c")
    iota = lax.iota(jnp.int32, 16)
    zeros16 = jnp.zeros((16,), jnp.int32)
    zf16 = jnp.zeros((16,), jnp.float32)

    gsems = ((0, sem_g0), (1, sem_g1), (2, sem_g2), (3, sem_g3), (4, sem_g4), (5, sem_g5))

    def gslice(par):
        return gbig.at[pl.ds(pl.multiple_of(par * GCHUNK, GCHUNK), GCHUNK)]

    def issue_gather(c, d0, par, sem):
        pltpu.async_copy(
            cb_hbm.at[tok_v.at[pl.ds(pl.multiple_of(c * GCHUNK, GCHUNK),
                                     GCHUNK)],
                      pl.ds(d0, DBLK)],
            gslice(par), sem)

    def wait_gather(d0, par, sem):
        pltpu.make_async_copy(
            cb_hbm.at[tok_v.at[pl.ds(0, GCHUNK)], pl.ds(d0, DBLK)],
            gslice(par), sem).wait()

    def transpose_chunk(c, par, spar, len16):
        gbase = par * GCHUNK
        spar16 = zeros16 + spar

        def lb_loop(lb, _):
            row16 = (gbase + lb * 16) + iota
            col16 = (c * GCHUNK + lb * 16) + iota
            lmask = col16 < len16

            @plsc.parallel_loop(0, 16, 1, unroll=2)
            def k_loop(k):
                rot = (iota + k) & 15
                for dv in range(NDV):
                    dloc = dv * 16 + rot
                    vec = plsc.load_gather(gbig, [row16, dloc])
                    val = jnp.where(lmask, vec, zf16)
                    plsc.store_scatter(sbig, [spar16, dloc, col16], val)

            return 0

        lax.fori_loop(0, NLB, lb_loop, 0)

    def do_item(t, _):
        bi = t // NDBLK
        b = wid * BPW + bi
        d0 = pl.multiple_of((t % NDBLK) * DBLK, DBLK)
        d0_next = pl.multiple_of(((t + 1) % NDBLK) * DBLK, DBLK)

        lvec = tok_v[pl.ds(LEN_COL - LEN_COL % 16, 16)]
        len_s = lvec[LEN_COL % 16]
        len16 = zeros16 + len_s

        spar = t % 2
        for wp, sem_w in ((0, sem_w0), (1, sem_w1)):
            @pl.when(jnp.logical_and(spar == wp, t >= 2))
            def _():
                pltpu.make_async_copy(
                    sbig.at[wp], out_hbm.at[b, pl.ds(d0, DBLK), :],
                    sem_w).wait()

        def chunk_body(c, _):
            par = c % NBUF
            nxt3 = (c + 5) % NBUF
            @pl.when(jnp.logical_and(c + 5 < NCHUNK,
                                     (c + 5) * GCHUNK < len_s))
            def _():
                for gp, sem in gsems:
                    @pl.when(nxt3 == gp)
                    def _():
                        issue_gather(c + 5, d0, gp, sem)

            @pl.when(jnp.logical_and(
                c + 5 >= NCHUNK,
                jnp.logical_and((t + 1) % NDBLK != 0,
                                (c + 5 - NCHUNK) * GCHUNK < len_s)))
            def _():
                for gp, sem in gsems:
                    @pl.when(nxt3 == gp)
                    def _():
                        issue_gather(c + 5 - NCHUNK, d0_next, gp, sem)

            @pl.when(c * GCHUNK < len_s)
            def _():
                for gp, sem in gsems:
                    @pl.when(par == gp)
                    def _():
                        wait_gather(d0, gp, sem)
                transpose_chunk(c, par, spar, len16)

            @pl.when(jnp.logical_and(c * GCHUNK >= len_s, t % NDBLK < 2))
            def _():
                col0 = pl.multiple_of(c * GCHUNK, GCHUNK)

                def z_loop(d, _):
                    sbig[spar, d, pl.ds(col0, 16)] = zf16
                    sbig[spar, d, pl.ds(col0 + 16, 16)] = zf16
                    return 0

                lax.fori_loop(0, DBLK, z_loop, 0)
            return 0

        lax.fori_loop(0, NCHUNK, chunk_body, 0)

        for wp, sem_w in ((0, sem_w0), (1, sem_w1)):
            @pl.when(spar == wp)
            def _():
                pltpu.async_copy(
                    sbig.at[wp], out_hbm.at[b, pl.ds(d0, DBLK), :], sem_w)

        @pl.when(jnp.logical_and((t + 1) % NDBLK == 0, t + 1 < ITEMS))
        def _():
            pltpu.sync_copy(tok_hbm.at[b + 1], tok_v)
            lv2 = tok_v[pl.ds(LEN_COL - LEN_COL % 16, 16)]
            ln2 = lv2[LEN_COL % 16]
            for gi, (gp, sem) in enumerate(gsems[:NBUF - 1]):
                @pl.when(gi * GCHUNK < ln2)
                def _():
                    issue_gather(gi, 0, gp, sem)
        return 0

    pltpu.sync_copy(tok_hbm.at[wid * BPW], tok_v)
    lv0 = tok_v[pl.ds(LEN_COL - LEN_COL % 16, 16)]
    ln0 = lv0[LEN_COL % 16]
    for gi, (gp, sem) in enumerate(gsems[:NBUF - 1]):
        @pl.when(gi * GCHUNK < ln0)
        def _():
            issue_gather(gi, 0, gp, sem)
    lax.fori_loop(0, ITEMS, do_item, 0)

    b_last = wid * BPW + BPW - 1
    pltpu.make_async_copy(
        sbig.at[0], out_hbm.at[b_last, pl.ds(0, DBLK), :], sem_w0).wait()
    pltpu.make_async_copy(
        sbig.at[1], out_hbm.at[b_last, pl.ds(0, DBLK), :], sem_w1).wait()


@functools.partial(jax.jit, donate_argnums=())
def _run(codebook, tokens_pad):
    mesh = plsc.VectorSubcoreMesh(core_axis_name="c", subcore_axis_name="s")
    k = pl.kernel(
        _body,
        out_type=jax.ShapeDtypeStruct((B, D, L), jnp.float32),
        mesh=mesh,
        compiler_params=pltpu.CompilerParams(
            use_tc_tiling_on_sc=True, needs_layout_passes=False),
        scratch_types=[
            pltpu.VMEM((LPAD,), jnp.int32),
            pltpu.VMEM((NBUF * GCHUNK, DBLK), jnp.float32),
            pltpu.VMEM((2, DBLK, L), jnp.float32),
            pltpu.SemaphoreType.DMA,
            pltpu.SemaphoreType.DMA,
            pltpu.SemaphoreType.DMA,
            pltpu.SemaphoreType.DMA,
            pltpu.SemaphoreType.DMA,
            pltpu.SemaphoreType.DMA,
            pltpu.SemaphoreType.DMA,
            pltpu.SemaphoreType.DMA,
        ],
    )
    return k(codebook, tokens_pad)


def kernel(audio_tokens, output_lengths, codebook):
    tokens_pad = jnp.pad(audio_tokens, ((0, 0), (0, LPAD - L)))
    tokens_pad = tokens_pad.at[:, LEN_COL].set(output_lengths)
    out = _run(codebook, tokens_pad)
    return (out, output_lengths)
